# baseline (speedup 1.0000x reference)
import functools

import jax
import jax.numpy as jnp
from jax import lax
from jax.experimental import pallas as pl
from jax.experimental.pallas import tpu as pltpu
from jax.experimental.pallas import tpu_sc as plsc

BATCH = 4096
HIST = 50
D = 64
NC = 2
NS = 16
NW = NC * NS
BPW = BATCH // NW
CB = 2
ROWS = CB * HIST
NCHUNK = BPW // CB
LANES = 16
KPR = D // LANES

_sc_mesh = plsc.VectorSubcoreMesh(core_axis_name="c", subcore_axis_name="s")


@functools.partial(
    pl.kernel,
    out_type=jax.ShapeDtypeStruct((BATCH, D), jnp.float32),
    mesh=_sc_mesh,
    scratch_types=[
        pltpu.VMEM((NCHUNK, ROWS), jnp.int32),
        pltpu.VMEM((ROWS, D), jnp.float32),
        pltpu.VMEM((BPW, D), jnp.float32),
        pltpu.SemaphoreType.DMA,
    ],
    compiler_params=pltpu.CompilerParams(use_tc_tiling_on_sc=False),
)
def _pool(x2d_hbm, table_hbm, out_hbm, idx_v, buf_v, acc_v, sem):
    wid = lax.axis_index("s") * NC + lax.axis_index("c")
    pltpu.sync_copy(x2d_hbm.at[pl.ds(wid * NCHUNK, NCHUNK)], idx_v)

    def chunk_body(g, carry):
        pltpu.async_copy(table_hbm.at[idx_v.at[g]], buf_v, sem).wait()
        for c in range(CB):
            for k in range(KPR):
                def lane_body(l, a):
                    return a + buf_v[c * HIST + l, pl.ds(k * LANES, LANES)]
                a = lax.fori_loop(0, HIST, lane_body,
                                  jnp.zeros((LANES,), jnp.float32))
                acc_v[g * CB + c, pl.ds(k * LANES, LANES)] = a
        return carry

    lax.fori_loop(0, NCHUNK, chunk_body, 0)
    pltpu.sync_copy(acc_v, out_hbm.at[pl.ds(wid * BPW, BPW)])


def _linear_body(p_ref, w_ref, b_ref, o_ref):
    pooled = p_ref[...] * (1.0 / HIST)
    o_ref[...] = lax.dot_general(
        pooled, w_ref[...],
        dimension_numbers=(((1,), (1,)), ((), ())),
        preferred_element_type=jnp.float32,
    ) + b_ref[...]


_linear = pl.pallas_call(
    _linear_body,
    out_shape=jax.ShapeDtypeStruct((BATCH, D), jnp.float32),
)


def kernel(x, table, W, b):
    x2d = x.reshape(BATCH * HIST // ROWS, ROWS)
    sums = _pool(x2d, table)
    return _linear(sums, W, b.reshape(1, D))


# speedup vs baseline: 1.1609x; 1.1609x over previous
import functools

import jax
import jax.numpy as jnp
from jax import lax
from jax.experimental import pallas as pl
from jax.experimental.pallas import tpu as pltpu
from jax.experimental.pallas import tpu_sc as plsc

BATCH = 4096
HIST = 50
D = 64
NC = 2
NS = 16
NW = NC * NS
BPW = BATCH // NW

_sc_mesh = plsc.VectorSubcoreMesh(core_axis_name="c", subcore_axis_name="s")


@functools.partial(
    pl.kernel,
    out_type=jax.ShapeDtypeStruct((BATCH, D), jnp.float32),
    mesh=_sc_mesh,
    scratch_types=[
        pltpu.VMEM((HIST, BPW), jnp.int32),
        pltpu.VMEM((BPW, D), jnp.float32),
        pltpu.SemaphoreType.DMA,
    ],
    compiler_params=pltpu.CompilerParams(use_tc_tiling_on_sc=False),
)
def _pool(xt3_hbm, table_hbm, out_hbm, idx_v, acc_v, sem):
    wid = lax.axis_index("s") * NC + lax.axis_index("c")
    pltpu.sync_copy(xt3_hbm.at[:, wid, :], idx_v)
    pltpu.async_copy(table_hbm.at[idx_v.at[0]], acc_v, sem).wait()

    def lbody(l, carry):
        pltpu.async_copy(table_hbm.at[idx_v.at[l]], acc_v, sem, add=True).wait()
        return carry

    lax.fori_loop(1, HIST, lbody, 0)
    pltpu.sync_copy(acc_v, out_hbm.at[pl.ds(wid * BPW, BPW)])


def _linear_body(p_ref, w_ref, b_ref, o_ref):
    pooled = p_ref[...] * (1.0 / HIST)
    o_ref[...] = lax.dot_general(
        pooled, w_ref[...],
        dimension_numbers=(((1,), (1,)), ((), ())),
        preferred_element_type=jnp.float32,
    ) + b_ref[...]


_linear = pl.pallas_call(
    _linear_body,
    out_shape=jax.ShapeDtypeStruct((BATCH, D), jnp.float32),
)


def kernel(x, table, W, b):
    xt3 = x.T.reshape(HIST, NW, BPW)
    sums = _pool(xt3, table)
    return _linear(sums, W, b.reshape(1, D))


# speedup vs baseline: 1.2150x; 1.0466x over previous
import functools

import jax
import jax.numpy as jnp
from jax import lax
from jax.experimental import pallas as pl
from jax.experimental.pallas import tpu as pltpu
from jax.experimental.pallas import tpu_sc as plsc

BATCH = 4096
HIST = 50
D = 64
NC = 2
NS = 16
NW = NC * NS
BPW = BATCH // NW

_sc_mesh = plsc.VectorSubcoreMesh(core_axis_name="c", subcore_axis_name="s")


@functools.partial(
    pl.kernel,
    out_type=jax.ShapeDtypeStruct((BATCH, D), jnp.float32),
    mesh=_sc_mesh,
    scratch_types=[
        pltpu.VMEM((HIST, BPW), jnp.int32),
        pltpu.VMEM((BPW, D), jnp.float32),
        pltpu.SemaphoreType.DMA,
        pltpu.SemaphoreType.DMA,
    ],
    compiler_params=pltpu.CompilerParams(use_tc_tiling_on_sc=False),
)
def _pool(xt3_hbm, table_hbm, out_hbm, idx_v, acc_v, sem_i, sem_g):
    wid = lax.axis_index("s") * NC + lax.axis_index("c")
    idx_cp = pltpu.async_copy(xt3_hbm.at[:, wid, :], idx_v, sem_i)

    zero = jnp.zeros((16,), jnp.float32)

    def zbody(i, carry):
        for k in range(D // 16):
            acc_v[i, pl.ds(k * 16, 16)] = zero
        return carry

    lax.fori_loop(0, BPW, zbody, 0)
    idx_cp.wait()

    def fire(l, carry):
        pltpu.async_copy(table_hbm.at[idx_v.at[l]], acc_v, sem_g, add=True)
        return carry

    lax.fori_loop(0, HIST, fire, 0)

    def drain(l, carry):
        pltpu.make_async_copy(table_hbm.at[idx_v.at[0]], acc_v, sem_g).wait()
        return carry

    lax.fori_loop(0, HIST, drain, 0)
    pltpu.sync_copy(acc_v, out_hbm.at[pl.ds(wid * BPW, BPW)])


def _linear_body(p_ref, w_ref, b_ref, o_ref):
    pooled = p_ref[...] * (1.0 / HIST)
    o_ref[...] = lax.dot_general(
        pooled, w_ref[...],
        dimension_numbers=(((1,), (1,)), ((), ())),
        preferred_element_type=jnp.float32,
    ) + b_ref[...]


_linear = pl.pallas_call(
    _linear_body,
    out_shape=jax.ShapeDtypeStruct((BATCH, D), jnp.float32),
)


def kernel(x, table, W, b):
    xt3 = x.T.reshape(HIST, NW, BPW)
    sums = _pool(xt3, table)
    return _linear(sums, W, b.reshape(1, D))


# speedup vs baseline: 1.2170x; 1.0017x over previous
import functools

import jax
import jax.numpy as jnp
from jax import lax
from jax.experimental import pallas as pl
from jax.experimental.pallas import tpu as pltpu
from jax.experimental.pallas import tpu_sc as plsc

BATCH = 4096
HIST = 50
D = 64
NC = 2
NS = 16
NW = NC * NS
BPW = BATCH // NW
LANES = 16

_sc_mesh = plsc.VectorSubcoreMesh(core_axis_name="c", subcore_axis_name="s")


@functools.partial(
    pl.kernel,
    out_type=jax.ShapeDtypeStruct((BATCH, D), jnp.float32),
    mesh=_sc_mesh,
    scratch_types=[
        pltpu.VMEM((HIST, BPW), jnp.int32),
        pltpu.VMEM((BPW, D), jnp.float32),
        pltpu.SemaphoreType.DMA,
        pltpu.SemaphoreType.DMA,
    ],
    compiler_params=pltpu.CompilerParams(use_tc_tiling_on_sc=False),
)
def _pool(xt3_hbm, table_hbm, out_hbm, idx_v, acc_v, sem_i, sem_g):
    wid = lax.axis_index("s") * NC + lax.axis_index("c")
    idx_cp = pltpu.async_copy(xt3_hbm.at[:, wid, :], idx_v, sem_i)

    zero = jnp.zeros((LANES,), jnp.float32)

    def zbody(i, carry):
        for k in range(D // LANES):
            acc_v[i, pl.ds(k * LANES, LANES)] = zero
        return carry

    lax.fori_loop(0, BPW, zbody, 0)
    idx_cp.wait()

    def fire(l, carry):
        pltpu.async_copy(table_hbm.at[idx_v.at[l]], acc_v, sem_g, add=True)
        return carry

    lax.fori_loop(0, HIST, fire, 0)

    def drain(l, carry):
        pltpu.make_async_copy(table_hbm.at[idx_v.at[0]], acc_v, sem_g).wait()
        return carry

    lax.fori_loop(0, HIST, drain, 0)
    pltpu.sync_copy(acc_v, out_hbm.at[pl.ds(wid * BPW, BPW)])


def _tr_body(x_ref, o_ref):
    o_ref[...] = x_ref[...].T


_transpose = pl.pallas_call(
    _tr_body,
    out_shape=jax.ShapeDtypeStruct((HIST, BATCH), jnp.int32),
)


def _linear_body(p_ref, w_ref, b_ref, o_ref):
    pooled = p_ref[...] * (1.0 / HIST)
    o_ref[...] = lax.dot_general(
        pooled, w_ref[...],
        dimension_numbers=(((1,), (1,)), ((), ())),
        preferred_element_type=jnp.float32,
    ) + b_ref[...]


_linear = pl.pallas_call(
    _linear_body,
    out_shape=jax.ShapeDtypeStruct((BATCH, D), jnp.float32),
)


def kernel(x, table, W, b):
    xt3 = _transpose(x).reshape(HIST, NW, BPW)
    sums = _pool(xt3, table)
    return _linear(sums, W, b.reshape(1, D))
